# baseline (speedup 1.0000x reference)
import jax
import jax.numpy as jnp
from jax.experimental import pallas as pl

_BB = 16


def _emb_body(rand_ref, col_ref, row_ref):
    r = rand_ref[...]
    n = r.shape[1]
    less = r[:, :, None] < r[:, None, :]
    kk = jax.lax.broadcasted_iota(jnp.int32, (1, n, n), 1)
    jj = jax.lax.broadcasted_iota(jnp.int32, (1, n, n), 2)
    tie = (r[:, :, None] == r[:, None, :]) & (kk < jj)
    rank = jnp.sum((less | tie).astype(jnp.int32), axis=1)
    ii = jax.lax.broadcasted_iota(jnp.int32, (1, n, n), 1)
    col_ref[...] = (rank[:, None, :] == ii).astype(col_ref.dtype)
    row_ref[...] = jnp.zeros_like(row_ref)


def kernel(cost_matrix):
    b, n, _ = cost_matrix.shape
    rkey = jax.random.fold_in(jax.random.key(0), 1)
    rand = jax.random.uniform(rkey, (b, n), dtype=jnp.float32)
    col_emb, row_emb = pl.pallas_call(
        _emb_body,
        grid=(b // _BB,),
        in_specs=[pl.BlockSpec((_BB, n), lambda i: (i, 0))],
        out_specs=[
            pl.BlockSpec((_BB, n, n), lambda i: (i, 0, 0)),
            pl.BlockSpec((_BB, n, n), lambda i: (i, 0, 0)),
        ],
        out_shape=[
            jax.ShapeDtypeStruct((b, n, n), cost_matrix.dtype),
            jax.ShapeDtypeStruct((b, n, n), cost_matrix.dtype),
        ],
    )(rand)
    return (row_emb, col_emb, cost_matrix)


# speedup vs baseline: 1.0043x; 1.0043x over previous
import jax
import jax.numpy as jnp
from jax.experimental import pallas as pl

_BB = 8


def _emb_body(rand_ref, col_ref, row_ref):
    r = rand_ref[...]
    n = r.shape[1]
    less = r[:, :, None] < r[:, None, :]
    kk = jax.lax.broadcasted_iota(jnp.int32, (1, n, n), 1)
    jj = jax.lax.broadcasted_iota(jnp.int32, (1, n, n), 2)
    tie = (r[:, :, None] == r[:, None, :]) & (kk < jj)
    rank = jnp.sum((less | tie).astype(jnp.int32), axis=1)
    ii = jax.lax.broadcasted_iota(jnp.int32, (1, n, n), 1)
    col_ref[...] = (rank[:, None, :] == ii).astype(col_ref.dtype)
    row_ref[...] = jnp.zeros_like(row_ref)


def kernel(cost_matrix):
    b, n, _ = cost_matrix.shape
    rkey = jax.random.fold_in(jax.random.key(0), 1)
    rand = jax.random.uniform(rkey, (b, n), dtype=jnp.float32)
    col_emb, row_emb = pl.pallas_call(
        _emb_body,
        grid=(b // _BB,),
        in_specs=[pl.BlockSpec((_BB, n), lambda i: (i, 0))],
        out_specs=[
            pl.BlockSpec((_BB, n, n), lambda i: (i, 0, 0)),
            pl.BlockSpec((_BB, n, n), lambda i: (i, 0, 0)),
        ],
        out_shape=[
            jax.ShapeDtypeStruct((b, n, n), cost_matrix.dtype),
            jax.ShapeDtypeStruct((b, n, n), cost_matrix.dtype),
        ],
    )(rand)
    return (row_emb, col_emb, cost_matrix)
